# baseline (speedup 1.0000x reference)
import jax
import jax.numpy as jnp
from jax.experimental import pallas as pl
from jax.experimental.pallas import tpu as pltpu

N = 1024
D = 64
K = 3
NL = 2
BS = 64
BT = 2

_F32 = jnp.float32
_BF16 = jnp.bfloat16


def _body(inp_ref, hx_ref, sb_ref,
          wx0_ref, wh2_0_ref, wh1_0_ref, gb0_ref, W0_ref, b0_ref, R0_ref, aw0_ref,
          wx1_ref, wh2_1_ref, wh1_1_ref, gb1_ref, W1_ref, b1_ref, R1_ref, aw1_ref,
          out_ref, hxn_ref):
    Sb = sb_ref[...]

    def layer(Tin, l, wh2_ref, wh1_ref, gb_ref, W_ref, b_ref, R_ref, aw_ref):
        H2 = jnp.concatenate([hx_ref[l, e, 2] for e in range(BT)], axis=0)
        H1 = jnp.concatenate([hx_ref[l, e, 1] for e in range(BT)], axis=0)
        T = (Tin
             + jnp.dot(H2.astype(_BF16), wh2_ref[...], preferred_element_type=_F32)
             + jnp.dot(H1.astype(_BF16), wh1_ref[...], preferred_element_type=_F32))
        TCc = jnp.concatenate(
            [T[e * N:(e + 1) * N, 2 * D:3 * D] for e in range(BT)], axis=1)
        U = jnp.dot(Sb, TCc.astype(_BF16), preferred_element_type=_F32)
        TBc = jnp.concatenate(
            [T[e * N:(e + 1) * N, D:2 * D] for e in range(BT)], axis=1) + U
        V = jnp.dot(Sb, TBc.astype(_BF16), preferred_element_type=_F32)
        convs = []
        for e in range(BT):
            diff = T[e * N:(e + 1) * N, 0:D] + V[:, e * D:(e + 1) * D] + gb_ref[...]
            convs.append(jnp.where(diff >= 0, diff, 0.01 * diff))
        CONV = jnp.dot(jnp.concatenate(convs, axis=0).astype(_BF16), W_ref[...],
                       preferred_element_type=_F32)
        outs = []
        aw = aw_ref[...]
        for e in range(BT):
            ns0 = hx_ref[l, e, 0] + R_ref[0]
            ns1 = hx_ref[l, e, 1] + R_ref[1]
            ns2 = hx_ref[l, e, 2] + R_ref[2]
            s0 = jnp.sum(ns0 * aw)
            s1 = jnp.sum(ns1 * aw)
            s2 = jnp.sum(ns2 * aw)
            m = jnp.maximum(jnp.maximum(s0, s1), s2)
            e0 = jnp.exp(s0 - m)
            e1 = jnp.exp(s1 - m)
            e2 = jnp.exp(s2 - m)
            inv = 1.0 / (e0 + e1 + e2)
            att = (e0 * inv) * ns0 + (e1 * inv) * ns1 + (e2 * inv) * ns2
            outs.append(CONV[e * N:(e + 1) * N] + b_ref[...] + att)
        return outs

    t_in0 = []
    for e in range(BT):
        xin = inp_ref[e]
        t_in0.append(xin[:, 0:1] * wx0_ref[0:1, :] + xin[:, 1:2] * wx0_ref[1:2, :])
    out0s = layer(jnp.concatenate(t_in0, axis=0), 0, wh2_0_ref, wh1_0_ref,
                  gb0_ref, W0_ref, b0_ref, R0_ref, aw0_ref)
    Tin1 = jnp.dot(jnp.concatenate(out0s, axis=0).astype(_BF16), wx1_ref[...],
                   preferred_element_type=_F32)
    out1s = layer(Tin1, 1, wh2_1_ref, wh1_1_ref,
                  gb1_ref, W1_ref, b1_ref, R1_ref, aw1_ref)
    for e in range(BT):
        hxn_ref[0, e, 0] = hx_ref[0, e, 1]
        hxn_ref[0, e, 1] = hx_ref[0, e, 2]
        hxn_ref[0, e, 2] = out0s[e]
        hxn_ref[1, e, 0] = hx_ref[1, e, 1]
        hxn_ref[1, e, 1] = hx_ref[1, e, 2]
        hxn_ref[1, e, 2] = out1s[e]
        out_ref[e] = out1s[e]


def _prep(gw, in_dim):
    g = gw.reshape(in_dim + 2 * D, K, D)
    wa = jnp.concatenate([g[:, 0] - g[:, 2], g[:, 1], 2.0 * g[:, 2]], axis=1)
    return wa[:in_dim], wa[in_dim:in_dim + D], wa[in_dim + D:]


def kernel(inputs, hx_k, support, gw0, gb0, W0, b0, R0, attw0, attb0,
           gw1, gb1, W1, b1, R1, attw1, attb1):
    del attb0, attb1
    inp3 = inputs.reshape(BS, N, 2)
    Sb = support.astype(_BF16)
    wx0, wh2_0, wh1_0 = _prep(gw0, 2)
    wx1, wh2_1, wh1_1 = _prep(gw1, D)
    wh2_0, wh1_0 = wh2_0.astype(_BF16), wh1_0.astype(_BF16)
    wx1, wh2_1, wh1_1 = wx1.astype(_BF16), wh2_1.astype(_BF16), wh1_1.astype(_BF16)
    W0b, W1b = W0.astype(_BF16), W1.astype(_BF16)
    aw0 = attw0.reshape(N, D)
    aw1 = attw1.reshape(N, D)
    gb0r = gb0.reshape(1, D)
    gb1r = gb1.reshape(1, D)

    grid = (BS // BT,)
    const = lambda shape: pl.BlockSpec(shape, lambda i: (0,) * len(shape))
    in_specs = [
        pl.BlockSpec((BT, N, 2), lambda i: (i, 0, 0)),
        pl.BlockSpec((NL, BT, K, N, D), lambda i: (0, i, 0, 0, 0)),
        const((N, N)),
        const((2, K * D)), const((D, K * D)), const((D, K * D)),
        const((1, D)), const((D, D)), const((N, D)), const((K, N, D)), const((N, D)),
        const((D, K * D)), const((D, K * D)), const((D, K * D)),
        const((1, D)), const((D, D)), const((N, D)), const((K, N, D)), const((N, D)),
    ]
    out_specs = [
        pl.BlockSpec((BT, N, D), lambda i: (i, 0, 0)),
        pl.BlockSpec((NL, BT, K, N, D), lambda i: (0, i, 0, 0, 0)),
    ]
    out, hxn = pl.pallas_call(
        _body,
        grid=grid,
        in_specs=in_specs,
        out_specs=out_specs,
        out_shape=[
            jax.ShapeDtypeStruct((BS, N, D), _F32),
            jax.ShapeDtypeStruct((NL, BS, K, N, D), _F32),
        ],
        compiler_params=pltpu.CompilerParams(
            dimension_semantics=("parallel",),
        ),
    )(inp3, hx_k, Sb,
      wx0, wh2_0, wh1_0, gb0r, W0b, b0, R0, aw0,
      wx1, wh2_1, wh1_1, gb1r, W1b, b1, R1, aw1)
    return out.reshape(BS, N * D), hxn


# speedup vs baseline: 2.8351x; 2.8351x over previous
import jax
import jax.numpy as jnp
from jax.experimental import pallas as pl
from jax.experimental.pallas import tpu as pltpu

N = 1024
D = 64
K = 3
NL = 2
BS = 64
BT = 4

_F32 = jnp.float32
_BF16 = jnp.bfloat16


def _body(inp_ref, hx_ref, st_ref,
          wx0_ref, wc0_ref, gb0_ref, w0t_ref, b0_ref, r0_ref, aw0_ref,
          wc1_ref, gb1_ref, w1t_ref, b1_ref, r1_ref, aw1_ref,
          out_ref, hxn_ref):
    St = st_ref[...]

    def layer(Ts, l, gb_ref, Wt_ref, b_ref, R_ref, aw_ref):
        convs = []
        for e in range(BT):
            T = Ts[e]
            u = jnp.dot(T[2 * D:3 * D].astype(_BF16), St,
                        preferred_element_type=_F32)
            v = jnp.dot((T[D:2 * D] + u).astype(_BF16), St,
                        preferred_element_type=_F32)
            diff = T[0:D] + v + gb_ref[...]
            convs.append(jnp.where(diff >= 0, diff, 0.01 * diff))
        CONV = jnp.dot(Wt_ref[...],
                       jnp.concatenate(convs, axis=1).astype(_BF16),
                       preferred_element_type=_F32)
        outs = []
        aw = aw_ref[...]
        for e in range(BT):
            ns0 = hx_ref[l, e, 0] + R_ref[0]
            ns1 = hx_ref[l, e, 1] + R_ref[1]
            ns2 = hx_ref[l, e, 2] + R_ref[2]
            s0 = jnp.sum(ns0 * aw)
            s1 = jnp.sum(ns1 * aw)
            s2 = jnp.sum(ns2 * aw)
            m = jnp.maximum(jnp.maximum(s0, s1), s2)
            e0 = jnp.exp(s0 - m)
            e1 = jnp.exp(s1 - m)
            e2 = jnp.exp(s2 - m)
            inv = 1.0 / (e0 + e1 + e2)
            att = (e0 * inv) * ns0 + (e1 * inv) * ns1 + (e2 * inv) * ns2
            outs.append(CONV[:, e * N:(e + 1) * N] + b_ref[...] + att)
        return outs

    Ts0 = []
    for e in range(BT):
        xin = inp_ref[e]
        t_in = (wx0_ref[:, 0:1] * xin[0:1, :] + wx0_ref[:, 1:2] * xin[1:2, :])
        Hcat = jnp.concatenate([hx_ref[0, e, 2], hx_ref[0, e, 1]], axis=0)
        Ts0.append(t_in + jnp.dot(wc0_ref[...], Hcat.astype(_BF16),
                                  preferred_element_type=_F32))
    out0s = layer(Ts0, 0, gb0_ref, w0t_ref, b0_ref, r0_ref, aw0_ref)

    Ts1 = []
    for e in range(BT):
        Hcat = jnp.concatenate(
            [out0s[e], hx_ref[1, e, 2], hx_ref[1, e, 1]], axis=0)
        Ts1.append(jnp.dot(wc1_ref[...], Hcat.astype(_BF16),
                           preferred_element_type=_F32))
    out1s = layer(Ts1, 1, gb1_ref, w1t_ref, b1_ref, r1_ref, aw1_ref)

    for e in range(BT):
        hxn_ref[0, e, 0] = hx_ref[0, e, 1]
        hxn_ref[0, e, 1] = hx_ref[0, e, 2]
        hxn_ref[0, e, 2] = out0s[e]
        hxn_ref[1, e, 0] = hx_ref[1, e, 1]
        hxn_ref[1, e, 1] = hx_ref[1, e, 2]
        hxn_ref[1, e, 2] = out1s[e]
        out_ref[e] = out1s[e]


def _prep(gw, in_dim):
    g = gw.reshape(in_dim + 2 * D, K, D)
    wa = jnp.concatenate([g[:, 0] - g[:, 2], g[:, 1], 2.0 * g[:, 2]], axis=1).T
    return wa[:, :in_dim], wa[:, in_dim:in_dim + D], wa[:, in_dim + D:]


def kernel(inputs, hx_k, support, gw0, gb0, W0, b0, R0, attw0, attb0,
           gw1, gb1, W1, b1, R1, attw1, attb1):
    del attb0, attb1
    inp_t = inputs.reshape(BS, N, 2).transpose(0, 2, 1)
    hx_t = jnp.swapaxes(hx_k, 3, 4)
    St = jnp.swapaxes(support, 0, 1).astype(_BF16)
    wx0, wh2_0, wh1_0 = _prep(gw0, 2)
    wx1, wh2_1, wh1_1 = _prep(gw1, D)
    wc0 = jnp.concatenate([wh2_0, wh1_0], axis=1).astype(_BF16)
    wc1 = jnp.concatenate([wx1, wh2_1, wh1_1], axis=1).astype(_BF16)
    w0t = W0.T.astype(_BF16)
    w1t = W1.T.astype(_BF16)
    b0t = b0.T
    b1t = b1.T
    r0t = jnp.swapaxes(R0, 1, 2)
    r1t = jnp.swapaxes(R1, 1, 2)
    aw0 = attw0.reshape(N, D).T
    aw1 = attw1.reshape(N, D).T
    gb0r = gb0.reshape(D, 1)
    gb1r = gb1.reshape(D, 1)

    grid = (BS // BT,)
    const = lambda shape: pl.BlockSpec(shape, lambda i: (0,) * len(shape))
    in_specs = [
        pl.BlockSpec((BT, 2, N), lambda i: (i, 0, 0)),
        pl.BlockSpec((NL, BT, K, D, N), lambda i: (0, i, 0, 0, 0)),
        const((N, N)),
        const((K * D, 2)), const((K * D, 2 * D)),
        const((D, 1)), const((D, D)), const((D, N)), const((K, D, N)), const((D, N)),
        const((K * D, K * D)),
        const((D, 1)), const((D, D)), const((D, N)), const((K, D, N)), const((D, N)),
    ]
    out_specs = [
        pl.BlockSpec((BT, D, N), lambda i: (i, 0, 0)),
        pl.BlockSpec((NL, BT, K, D, N), lambda i: (0, i, 0, 0, 0)),
    ]
    out, hxn = pl.pallas_call(
        _body,
        grid=grid,
        in_specs=in_specs,
        out_specs=out_specs,
        out_shape=[
            jax.ShapeDtypeStruct((BS, D, N), _F32),
            jax.ShapeDtypeStruct((NL, BS, K, D, N), _F32),
        ],
        compiler_params=pltpu.CompilerParams(
            dimension_semantics=("parallel",),
        ),
    )(inp_t, hx_t, St,
      wx0, wc0, gb0r, w0t, b0t, r0t, aw0,
      wc1, gb1r, w1t, b1t, r1t, aw1)
    out_f = out.transpose(0, 2, 1).reshape(BS, N * D)
    hxn_f = jnp.swapaxes(hxn, 3, 4)
    return out_f, hxn_f
